# baseline (speedup 1.0000x reference)
import functools

import jax
import jax.numpy as jnp
from jax import lax
from jax.experimental import pallas as pl
from jax.experimental.pallas import tpu as pltpu
from jax.experimental.pallas import tpu_sc as plsc

_B = 4096
_C = 9605
_CLIP = 0.05
_EPS = 1e-08
_ALPHA = 0.5

_BC = 256

_NC = 2
_NS = 16
_NW = _NC * _NS
_SPW = _B // _NW
_LN2 = 0.6931471805599453


def _elem_loss(x, y):
    t = jnp.tanh(x * 0.5)
    v = jnp.where(y > 0.5,
                  jnp.maximum(0.5 + 0.5 * t, _EPS),
                  jnp.minimum(0.55 - 0.5 * t, 1.0))
    return jnp.log(v)


def _total_body(x_ref, y_ref, out_ref):
    i = pl.program_id(0)
    total = jnp.sum(_elem_loss(x_ref[...], y_ref[...]))

    @pl.when(i == 0)
    def _init():
        out_ref[0, 0] = jnp.float32(0.0)

    out_ref[0, 0] += total


def _log_sc(v):
    bits = lax.bitcast_convert_type(v, jnp.int32)
    ex = jnp.float32(lax.shift_right_logical(bits, 23) & 0xFF) - 127.0
    m = lax.bitcast_convert_type((bits & 0x007FFFFF) | 0x3F800000,
                                 jnp.float32)
    big = m > 1.4142135
    m = jnp.where(big, 0.5 * m, m)
    ex = jnp.where(big, ex + 1.0, ex)
    u = m - 1.0
    p = jnp.float32(-1.0 / 8.0)
    for k in (7, 6, 5, 4, 3, 2, 1):
        p = p * u + jnp.float32((-1.0) ** (k + 1) / k)
    return ex * jnp.float32(_LN2) + p * u


def _loss_sc(xv):
    e = jnp.exp(-xv)
    s = 1.0 / (1.0 + e)
    v = jnp.minimum(1.0 + jnp.float32(_CLIP) - s, jnp.float32(1.0))
    return _log_sc(v)


def _corr_body(x_hbm, y_hbm, out_hbm, xv, yv, accv, sem):
    wid = lax.axis_index("s") * _NC + lax.axis_index("c")
    base = wid * _SPW
    _EXPERIMENT_NOOP = True

    def chunk(c, acc):
        col = c * 16
        xs = [xv[r, pl.ds(col, 16)] for r in range(12)]
        ys = [yv[r, pl.ds(col, 16)] for r in range(12)]
        ls = [_loss_sc(xs[r]) for r in range(12)]
        s_r = ys[0] + ys[1] + ys[2] + ys[3] + ys[4]
        s_d = ys[5] + ys[6] + ys[7] + ys[8]
        s_c = ys[9] + ys[10] + ys[11]
        l_r = ls[0] + ls[1] + ls[2] + ls[3] + ls[4]
        l_d = ls[5] + ls[6] + ls[7] + ls[8]
        l_c = ls[9] + ls[10] + ls[11]
        any_active = (s_r > 0.0) | (s_d > 0.0) | (s_c > 0.0)
        zero = jnp.zeros((16,), jnp.float32)
        inactive = (jnp.where(s_r == 0.0, l_r, zero)
                    + jnp.where(s_d == 0.0, l_d, zero)
                    + jnp.where(s_c == 0.0, l_c, zero))
        return acc + jnp.where(any_active, inactive, zero)

    accv[...] = jnp.zeros((16,), jnp.float32)
    pltpu.sync_copy(accv, out_hbm.at[wid])


@jax.jit
def kernel(x, y):
    xt = x.T
    yt = y.T

    mesh = plsc.VectorSubcoreMesh(core_axis_name="c", subcore_axis_name="s")
    corr_parts = pl.kernel(
        _corr_body,
        mesh=mesh,
        out_type=jax.ShapeDtypeStruct((_NW, 16), jnp.float32),
        scratch_types=[
            pltpu.VMEM((16, _SPW), jnp.float32),
            pltpu.VMEM((16, _SPW), jnp.float32),
            pltpu.VMEM((16,), jnp.float32),
            pltpu.SemaphoreType.DMA,
        ],
    )(xt, yt)

    total = pl.pallas_call(
        _total_body,
        grid=(_B // _BC,),
        in_specs=[
            pl.BlockSpec((_C, _BC), lambda i: (0, i)),
            pl.BlockSpec((_C, _BC), lambda i: (0, i)),
        ],
        out_specs=pl.BlockSpec((1, 1), lambda i: (0, 0),
                               memory_space=pltpu.SMEM),
        out_shape=jax.ShapeDtypeStruct((1, 1), jnp.float32),
    )(xt, yt)

    corr = jnp.sum(corr_parts)
    return -(total[0, 0] - (1.0 - _ALPHA) * corr)


# speedup vs baseline: 1.1652x; 1.1652x over previous
import jax
import jax.numpy as jnp
from jax.experimental import pallas as pl
from jax.experimental.pallas import tpu as pltpu

_B = 4096
_C = 9605
_CLIP = 0.05
_EPS = 1e-08
_ALPHA = 0.5

_BC = 256


def _elem_loss(x, y):
    t = jnp.tanh(x * 0.5)
    v = jnp.where(y > 0.5,
                  jnp.maximum(0.5 + 0.5 * t, _EPS),
                  jnp.minimum(0.55 - 0.5 * t, 1.0))
    return jnp.log(v)


def _loss_body(x_ref, y_ref, out_ref):
    i = pl.program_id(0)
    loss = _elem_loss(x_ref[...], y_ref[...])
    total = jnp.sum(loss)

    x12 = x_ref[0:12, :]
    y12 = y_ref[0:12, :]
    l12 = _elem_loss(x12, y12)
    s_r = jnp.sum(y12[0:5, :], axis=0)
    s_d = jnp.sum(y12[5:9, :], axis=0)
    s_c = jnp.sum(y12[9:12, :], axis=0)
    L_r = jnp.sum(l12[0:5, :], axis=0)
    L_d = jnp.sum(l12[5:9, :], axis=0)
    L_c = jnp.sum(l12[9:12, :], axis=0)
    any_active = (s_r > 0) | (s_d > 0) | (s_c > 0)
    inactive_loss = (jnp.where(s_r == 0, L_r, 0.0)
                     + jnp.where(s_d == 0, L_d, 0.0)
                     + jnp.where(s_c == 0, L_c, 0.0))
    corr = jnp.sum(jnp.where(any_active, inactive_loss, 0.0))

    blk = total - (1.0 - _ALPHA) * corr

    @pl.when(i == 0)
    def _init():
        out_ref[0, 0] = jnp.float32(0.0)

    out_ref[0, 0] += blk


@jax.jit
def kernel(x, y):
    xt = x.T
    yt = y.T
    out = pl.pallas_call(
        _loss_body,
        grid=(_B // _BC,),
        in_specs=[
            pl.BlockSpec((_C, _BC), lambda i: (0, i)),
            pl.BlockSpec((_C, _BC), lambda i: (0, i)),
        ],
        out_specs=pl.BlockSpec((1, 1), lambda i: (0, 0),
                               memory_space=pltpu.SMEM),
        out_shape=jax.ShapeDtypeStruct((1, 1), jnp.float32),
    )(xt, yt)
    return -out[0, 0]


# speedup vs baseline: 1.1780x; 1.0110x over previous
import jax
import jax.numpy as jnp
from jax.experimental import pallas as pl
from jax.experimental.pallas import tpu as pltpu

_B = 4096
_C = 9605
_CLIP = 0.05
_EPS = 1e-08
_ALPHA = 0.5

_BC = 256


def _elem_loss(x, y):
    t = jnp.tanh(x * 0.5)
    v = jnp.where(y > 0.5,
                  jnp.maximum(0.5 + 0.5 * t, _EPS),
                  jnp.minimum(0.55 - 0.5 * t, 1.0))
    return jnp.log(v)


def _loss_body(x_ref, y_ref, out_ref):
    i = pl.program_id(0)
    loss = _elem_loss(x_ref[...], y_ref[...])
    total = jnp.sum(loss)

    x12 = x_ref[0:12, :]
    y12 = y_ref[0:12, :]
    l12 = _elem_loss(x12, y12)
    s_r = jnp.sum(y12[0:5, :], axis=0)
    s_d = jnp.sum(y12[5:9, :], axis=0)
    s_c = jnp.sum(y12[9:12, :], axis=0)
    L_r = jnp.sum(l12[0:5, :], axis=0)
    L_d = jnp.sum(l12[5:9, :], axis=0)
    L_c = jnp.sum(l12[9:12, :], axis=0)
    any_active = (s_r > 0) | (s_d > 0) | (s_c > 0)
    inactive_loss = (jnp.where(s_r == 0, L_r, 0.0)
                     + jnp.where(s_d == 0, L_d, 0.0)
                     + jnp.where(s_c == 0, L_c, 0.0))
    corr = jnp.sum(jnp.where(any_active, inactive_loss, 0.0))

    blk = (1.0 - _ALPHA) * corr - total

    @pl.when(i == 0)
    def _init():
        out_ref[0, 0] = jnp.float32(0.0)

    out_ref[0, 0] += blk


@jax.jit
def kernel(x, y):
    xt = x.T
    yt = y.T
    out = pl.pallas_call(
        _loss_body,
        grid=(_B // _BC,),
        in_specs=[
            pl.BlockSpec((_C, _BC), lambda i: (0, i)),
            pl.BlockSpec((_C, _BC), lambda i: (0, i)),
        ],
        out_specs=pl.BlockSpec((1, 1), lambda i: (0, 0),
                               memory_space=pltpu.SMEM),
        out_shape=jax.ShapeDtypeStruct((1, 1), jnp.float32),
    )(xt, yt)
    return out[0, 0]
